# Initial kernel scaffold; baseline (speedup 1.0000x reference)
import jax
import jax.numpy as jnp
from jax.experimental import pallas as pl


def kernel(h, edge_index, pathway_ids, params):
    raise NotImplementedError("write your pallas kernel here")


# speedup vs baseline: 14.7399x; 14.7399x over previous
import functools
import math

import jax
import jax.numpy as jnp
from jax import lax
from jax.experimental import pallas as pl
from jax.experimental.pallas import tpu as pltpu
from jax.experimental.pallas import tpu_sc as plsc

N = 10000
E = 320000
IN_DIM = 16
H = 128
P = 8

NT = 10240
TILES = 32
EPT = E // TILES
CH = 128
L = ((EPT + CH - 1) // CH) * CH
N16 = N + 16
RP = NT // 16
K1 = int(math.ceil(0.8 * N))
K2 = int(math.ceil(0.8 * K1))
K3 = int(math.ceil(0.8 * K2))
INT_MIN = -2147483648

_mesh = plsc.VectorSubcoreMesh(core_axis_name="c", subcore_axis_name="s")
_sc_params = pltpu.CompilerParams(use_tc_tiling_on_sc=False,
                                  needs_layout_passes=False)


def _make_agg(D):
    @functools.partial(
        pl.kernel,
        out_type=jax.ShapeDtypeStruct((2, NT, D), jnp.float32),
        mesh=_mesh,
        compiler_params=_sc_params,
        scratch_types=[
            pltpu.VMEM((CH,), jnp.int32),
            pltpu.VMEM((CH,), jnp.int32),
            pltpu.VMEM((CH, D), jnp.float32),
            pltpu.VMEM((16,), jnp.int32),
            pltpu.VMEM_SHARED((NT, D), jnp.float32),
            pltpu.SemaphoreType.DMA,
        ],
    )
    def agg(table, srcs, dsts, counts, zrows, out, srcb, dstb, rows, cbuf,
            aggs, sem):
        c = lax.axis_index("c")
        s = lax.axis_index("s")
        tile = c * 16 + s
        pltpu.sync_copy(zrows, rows)
        for i in range(RP // CH):
            pltpu.sync_copy(rows, aggs.at[pl.ds(s * RP + i * CH, CH)])
        plsc.subcore_barrier()
        pltpu.sync_copy(counts.at[tile], cbuf)
        cnt = cbuf[...][0]
        nch = (cnt + (CH - 1)) // CH

        def body(i, carry):
            pltpu.sync_copy(srcs.at[tile, pl.ds(i * CH, CH)], srcb)
            pltpu.sync_copy(dsts.at[tile, pl.ds(i * CH, CH)], dstb)
            pltpu.async_copy(table.at[srcb], rows, sem).wait()
            pltpu.sync_copy(rows, aggs.at[dstb], add=True)
            return carry

        lax.fori_loop(0, nch, body, jnp.int32(0))
        plsc.subcore_barrier()
        pltpu.sync_copy(aggs.at[pl.ds(s * RP, RP)], out.at[c, pl.ds(s * RP, RP)])

    return agg


@functools.partial(
    pl.kernel,
    out_type=(
        jax.ShapeDtypeStruct((TILES, L), jnp.int32),
        jax.ShapeDtypeStruct((TILES, L), jnp.int32),
        jax.ShapeDtypeStruct((TILES, 16), jnp.int32),
        jax.ShapeDtypeStruct((2, NT, 16), jnp.float32),
    ),
    mesh=_mesh,
    compiler_params=_sc_params,
    scratch_types=[
        pltpu.VMEM((N16,), jnp.int32),
        pltpu.VMEM((L,), jnp.int32),
        pltpu.VMEM((L,), jnp.int32),
        pltpu.VMEM((L,), jnp.int32),
        pltpu.VMEM((L,), jnp.int32),
        pltpu.VMEM((CH,), jnp.int32),
        pltpu.VMEM((CH, 16), jnp.float32),
        pltpu.VMEM((CH, 16), jnp.float32),
        pltpu.VMEM((16,), jnp.int32),
        pltpu.VMEM_SHARED((NT, 16), jnp.float32),
    ],
)
def _filt(srcs, dsts, pids, ones_in, zeros_in, srcf, dstf, counts, degf,
          pidsv, srcv, dstv, sfv, dfv, dstb, onesb, zb, cntb, degs):
    c = lax.axis_index("c")
    s = lax.axis_index("s")
    tile = c * 16 + s
    pltpu.sync_copy(pids, pidsv)
    pltpu.sync_copy(srcs.at[tile], srcv)
    pltpu.sync_copy(dsts.at[tile], dstv)
    pltpu.sync_copy(ones_in, onesb)
    pltpu.sync_copy(zeros_in, zb)
    for i in range(RP // CH):
        pltpu.sync_copy(zb, degs.at[pl.ds(s * RP + i * CH, CH)])
    z16 = jnp.zeros((16,), jnp.int32)
    n16 = jnp.full((16,), N, jnp.int32)

    def pf(i, carry):
        sfv[pl.ds(i * 16, 16)] = z16
        dfv[pl.ds(i * 16, 16)] = n16
        return carry

    lax.fori_loop(0, L // 16, pf, jnp.int32(0))
    plsc.subcore_barrier()

    def cb(i, off):
        s16 = srcv[pl.ds(i * 16, 16)]
        d16 = dstv[pl.ds(i * 16, 16)]
        ps = plsc.load_gather(pidsv, [s16])
        pd = plsc.load_gather(pidsv, [d16])
        m = ps == pd
        plsc.store_compressed(sfv.at[pl.ds(off, 16)], s16, mask=m)
        plsc.store_compressed(dfv.at[pl.ds(off, 16)], d16, mask=m)
        pc = plsc.all_reduce_population_count(m)
        return off + pc[0]

    off = lax.fori_loop(0, L // 16, cb, jnp.int32(0))
    pltpu.sync_copy(sfv, srcf.at[tile])
    pltpu.sync_copy(dfv, dstf.at[tile])
    cntb[...] = jnp.full((16,), 1, jnp.int32) * off
    pltpu.sync_copy(cntb, counts.at[tile])

    def db(i, carry):
        pltpu.sync_copy(dstf.at[tile, pl.ds(i * CH, CH)], dstb)
        pltpu.sync_copy(onesb, degs.at[dstb], add=True)
        return carry

    lax.fori_loop(0, L // CH, db, jnp.int32(0))
    plsc.subcore_barrier()
    pltpu.sync_copy(degs.at[pl.ds(s * RP, RP)], degf.at[c, pl.ds(s * RP, RP)])


_agg32 = _make_agg(32)
_agg128 = _make_agg(128)
_agg144 = _make_agg(144)


def _rowdot(x, w):
    return jnp.sum(x * w[:, 0][None, :], axis=1)


def _gatt_global(x, w, b):
    g = _rowdot(x, w) + b[0]
    gm = jnp.max(g)
    gm = jnp.where(gm > -1e29, gm, 0.0)
    e = jnp.exp(g - gm)
    wts = e / jnp.clip(jnp.sum(e), 1e-16)
    return jnp.sum(wts[:, None] * x, axis=0)[None, :]


def _gatt_p(x, w, b, pids, nmask):
    oh = (pids[:, None] == lax.broadcasted_iota(jnp.int32, (1, P), 1)
          ).astype(jnp.float32)
    g = _rowdot(x, w) + b[0]
    g = jnp.where(nmask > 0, g, -1e30)
    gmax = jnp.max(jnp.where(oh > 0, g[:, None], -jnp.inf), axis=0)
    gmax = jnp.where(gmax > -1e29, gmax, 0.0)
    gpn = jnp.sum(oh * gmax[None, :], axis=1)
    e = jnp.exp(g - gpn) * nmask
    den = jnp.sum(oh * e[:, None], axis=0)
    dpn = jnp.sum(oh * den[None, :], axis=1)
    wts = e / jnp.clip(dpn, 1e-16)
    return lax.dot_general(oh, wts[:, None] * x, (((0,), (0,)), ((), ())),
                           preferred_element_type=jnp.float32)


def _gnorm(x, gnw, gnb, gna):
    mu = jnp.mean(x, axis=0)
    o = x - gna[None, :] * mu[None, :]
    var = jnp.mean(o * o, axis=0)
    return gnw[None, :] * o / jnp.sqrt(var + 1e-5)[None, :] + gnb[None, :]


def _f32key(x):
    i = lax.bitcast_convert_type(x, jnp.int32)
    return jnp.where(i >= 0, i, (~i) ^ jnp.int32(INT_MIN))


def _avg_ceil(lo, hi):
    return (lo >> 1) + (hi >> 1) + ((lo | hi) & 1)


def _topk_mask(sel, k):
    k2d = _f32key(sel).reshape(1, N)
    idx2d = lax.broadcasted_iota(jnp.int32, (1, N), 1)

    def b1(_, carry):
        lo, hi = carry
        mid = _avg_ceil(lo, hi)
        ok = jnp.sum((k2d >= mid).astype(jnp.int32)) >= k
        return jnp.where(ok, mid, lo), jnp.where(ok, hi, mid - 1)

    t, _ = lax.fori_loop(0, 32, b1, (jnp.int32(INT_MIN), jnp.int32(2147483647)))
    need = k - jnp.sum((k2d > t).astype(jnp.int32))

    def b2(_, carry):
        lo, hi = carry
        mid = _avg_ceil(lo, hi)
        ok = jnp.sum(((k2d == t) & (idx2d < mid)).astype(jnp.int32)) >= need
        return jnp.where(ok, lo, mid + 1), jnp.where(ok, mid, hi)

    _, m = lax.fori_loop(0, 15, b2, (jnp.int32(0), jnp.int32(N)))
    sel2d = (k2d > t) | ((k2d == t) & (idx2d < m))
    return sel2d.astype(jnp.float32).reshape(N)


_tc_params = pltpu.CompilerParams(vmem_limit_bytes=100 * 1024 * 1024)


def _tc(fn, out_shape, *args):
    return pl.pallas_call(fn, out_shape=out_shape,
                          compiler_params=_tc_params)(*args)


def _sage_head(agg_self, cnt, x, Wn, bn, Wr):
    mean = agg_self / cnt[:, None]
    return jnp.tanh(
        jnp.dot(mean, Wn, preferred_element_type=jnp.float32)
        + bn[None, :]
        + jnp.dot(x, Wr, preferred_element_type=jnp.float32))


def _tcA_body(agg, x_r, cnt_r, Wn, bn, Wr, gw, gb, gnw, gnb, gna,
              xg_o, xr_o, *, last):
    x = x_r[...]
    a = agg[0, :N, :] + agg[1, :N, :] + x
    xN = _sage_head(a, cnt_r[...], x, Wn[...], bn[...], Wr[...])
    xr_o[...] = _gatt_global(xN, gw[...], gb[...])
    if last:
        xg_o[...] = xN
    else:
        xg_o[...] = _gnorm(xN, gnw[...], gnb[...], gna[...])


def _tc1_body(agg, h_r, Wn, bn, Wr, gw, gb, gnw, gnb, gna, xg_o, xr_o, cnt_o):
    h = h_r[...]
    a = agg[0, :N, :] + agg[1, :N, :]
    cnt = a[:, 16] + 1.0
    aggH = a[:, :IN_DIM] + h
    xA = _sage_head(aggH, cnt, h, Wn[...], bn[...], Wr[...])
    xr_o[...] = _gatt_global(xA, gw[...], gb[...])
    xg_o[...] = _gnorm(xA, gnw[...], gnb[...], gna[...])
    cnt_o[...] = cnt


def _tc4_body(agg, degf, x_r, Wn, bn, Wr, x1B_o):
    x = x_r[...]
    a = agg[0, :N, :] + agg[1, :N, :] + x
    df = degf[0, :N, 0] + degf[1, :N, 0]
    cnt = jnp.clip(df + 1.0, 1.0, None)
    x1B_o[...] = _sage_head(a, cnt, x, Wn[...], bn[...], Wr[...])


def _pool_body(pagg, x_r, mask_prev, wrel, wroot, pb, gw, gb, pids, k,
               t_o, r_o):
    x = x_r[...]
    score = mask_prev * (_rowdot(pagg[0, :N, :], wrel)
                         + _rowdot(pagg[1, :N, :], wrel)
                         + _rowdot(x, wrel)) + _rowdot(x, wroot) + pb[0]
    sel = jnp.where(mask_prev > 0, score, -jnp.inf)
    newmask = _topk_mask(sel, k) * mask_prev
    xP = x * jnp.tanh(score)[:, None] * newmask[:, None]
    r_o[...] = _gatt_p(xP, gw, gb, pids, newmask)
    t_o[...] = jnp.concatenate(
        [xP, jnp.broadcast_to(newmask[:, None], (N, 16))], axis=1)


def _tc5_body(pagg, x_r, wrel, wroot, pb, gw, gb, pids_r, t_o, r_o):
    _pool_body(pagg, x_r, jnp.ones((N,), jnp.float32), wrel[...], wroot[...],
               pb[...], gw[...], gb[...], pids_r[...], K1, t_o, r_o)


def _tc7_body(pagg, x_r, t_r, wrel, wroot, pb, gw, gb, pids_r, t_o, r_o):
    _pool_body(pagg, x_r, t_r[:, H], wrel[...], wroot[...],
               pb[...], gw[...], gb[...], pids_r[...], K2, t_o, r_o)


def _sageB_body(agg, t_r, Wn, bn, Wr, x_o, tp_o):
    xP = t_r[:, :H]
    mprev = t_r[:, H]
    a = agg[0, :N, :] + agg[1, :N, :]
    aggx = mprev[:, None] * a[:, :H] + xP
    cnt = jnp.clip(mprev * a[:, H] + mprev, 1.0, None)
    xN = _sage_head(aggx, cnt, xP, Wn[...], bn[...], Wr[...])
    x_o[...] = xN
    tp_o[...] = xN * mprev[:, None]


def _tc9_body(pagg, x_r, t_r, wrel, wroot, pb, gw, gb, pids_r,
              x1, x2, x3, r1, r2, lin_w, lin_b, m1w, m1b, m2w, m2b, m3w, m3b,
              out_o):
    x = x_r[...]
    mprev = t_r[:, H]
    score = mprev * (_rowdot(pagg[0, :N, :], wrel[...])
                     + _rowdot(pagg[1, :N, :], wrel[...])
                     + _rowdot(x, wrel[...])) + _rowdot(x, wroot[...]) + pb[...][0]
    sel = jnp.where(mprev > 0, score, -jnp.inf)
    m3 = _topk_mask(sel, K3) * mprev
    xP3 = x * jnp.tanh(score)[:, None] * m3[:, None]
    r3 = _gatt_p(xP3, gw[...], gb[...], pids_r[...], m3)
    readout1 = jnp.concatenate([x1[...], x2[...], x3[...]], axis=1)
    readout2 = jnp.concatenate([r1[...], r2[...], r3], axis=1)
    readout = jnp.concatenate([readout1, readout2], axis=0)
    r = jnp.tanh(
        (jnp.dot(readout, lin_w[...], preferred_element_type=jnp.float32)
         + lin_b[...][None, :]).T)
    mm = jnp.tanh(jnp.dot(r, m1w[...], preferred_element_type=jnp.float32)
                  + m1b[...][None, :])
    mm = jnp.tanh(jnp.dot(mm, m2w[...], preferred_element_type=jnp.float32)
                  + m2b[...][None, :])
    mm = jax.nn.sigmoid(jnp.dot(mm, m3w[...], preferred_element_type=jnp.float32)
                        + m3b[...][None, :])
    out_o[...] = jax.nn.sigmoid(mm)


def kernel(h, edge_index, pathway_ids, params):
    f32 = jnp.float32
    pids = pathway_ids.astype(jnp.int32)
    pids_pad = jnp.concatenate([pids, jnp.full((16,), -1, jnp.int32)])
    src = edge_index[0].astype(jnp.int32).reshape(TILES, EPT)
    dst = edge_index[1].astype(jnp.int32).reshape(TILES, EPT)
    src2d = jnp.concatenate(
        [src, jnp.zeros((TILES, L - EPT), jnp.int32)], axis=1)
    dst2d = jnp.concatenate(
        [dst, jnp.full((TILES, L - EPT), N, jnp.int32)], axis=1)
    counts_full = jnp.full((TILES, 16), L, jnp.int32)
    ones16 = jnp.ones((CH, 16), f32)
    z16 = jnp.zeros((CH, 16), f32)
    z32 = jnp.zeros((CH, 32), f32)
    z128 = jnp.zeros((CH, H), f32)
    z144 = jnp.zeros((CH, H + 16), f32)
    table_hx = jnp.concatenate([h, jnp.ones((N, 16), f32)], axis=1)

    srcf, dstf, counts, degf = _filt(src2d, dst2d, pids_pad, ones16, z16)
    aggA = _agg32(table_hx, src2d, dst2d, counts_full, z32)

    p = params
    sN = jax.ShapeDtypeStruct((N, H), f32)
    s1 = jax.ShapeDtypeStruct((1, H), f32)
    sP = jax.ShapeDtypeStruct((P, H), f32)
    sT = jax.ShapeDtypeStruct((N, H + 16), f32)

    xg1, x1, cntA = _tc(
        _tc1_body, (sN, s1, jax.ShapeDtypeStruct((N,), f32)),
        aggA, h, p["conva_Wn"], p["conva_bn"], p["conva_Wr"],
        p["gatea_w"], p["gatea_b"], p["gn_w"], p["gn_b"], p["gn_a"])

    aggB = _agg128(xg1, src2d, dst2d, counts_full, z128)
    xg2, x2 = _tc(
        functools.partial(_tcA_body, last=False), (sN, s1),
        aggB, xg1, cntA, p["convb_Wn"], p["convb_bn"], p["convb_Wr"],
        p["gateb_w"], p["gateb_b"], p["gn_w"], p["gn_b"], p["gn_a"])

    aggC = _agg128(xg2, src2d, dst2d, counts_full, z128)
    xC, x3 = _tc(
        functools.partial(_tcA_body, last=True), (sN, s1),
        aggC, xg2, cntA, p["convc_Wn"], p["convc_bn"], p["convc_Wr"],
        p["gatec_w"], p["gatec_b"], p["gn_w"], p["gn_b"], p["gn_a"])

    agg1 = _agg128(xC, srcf, dstf, counts, z128)
    x1B = _tc(_tc4_body, sN, agg1, degf, xC,
              p["conv1_Wn"], p["conv1_bn"], p["conv1_Wr"])

    pagg1 = _agg128(x1B, srcf, dstf, counts, z128)
    t2, r1 = _tc(_tc5_body, (sT, sP), pagg1, x1B,
                 p["pool1_wrel"], p["pool1_wroot"], p["pool1_b"],
                 p["gate1_w"], p["gate1_b"], pids)

    agg2 = _agg144(t2, srcf, dstf, counts, z144)
    x2B, t2p = _tc(_sageB_body, (sN, sN), agg2, t2,
                   p["conv2_Wn"], p["conv2_bn"], p["conv2_Wr"])

    pagg2 = _agg128(t2p, srcf, dstf, counts, z128)
    t3, r2 = _tc(_tc7_body, (sT, sP), pagg2, x2B, t2,
                 p["pool2_wrel"], p["pool2_wroot"], p["pool2_b"],
                 p["gate2_w"], p["gate2_b"], pids)

    agg3 = _agg144(t3, srcf, dstf, counts, z144)
    x3B, t3p = _tc(_sageB_body, (sN, sN), agg3, t3,
                   p["conv3_Wn"], p["conv3_bn"], p["conv3_Wr"])

    pagg3 = _agg128(t3p, srcf, dstf, counts, z128)
    out = _tc(_tc9_body, jax.ShapeDtypeStruct((1, 1), f32),
              pagg3, x3B, t3,
              p["pool3_wrel"], p["pool3_wroot"], p["pool3_b"],
              p["gate3_w"], p["gate3_b"], pids,
              x1, x2, x3, r1, r2,
              p["lin_w"], p["lin_b"], p["mlp1_w"], p["mlp1_b"],
              p["mlp2_w"], p["mlp2_b"], p["mlp3_w"], p["mlp3_b"])
    return out


# speedup vs baseline: 18.1899x; 1.2341x over previous
import functools
import math

import jax
import jax.numpy as jnp
from jax import lax
from jax.experimental import pallas as pl
from jax.experimental.pallas import tpu as pltpu
from jax.experimental.pallas import tpu_sc as plsc

N = 10000
E = 320000
IN_DIM = 16
H = 128
P = 8

NT = 10240
TILES = 32
EPT = E // TILES
CH = 128
L = ((EPT + CH - 1) // CH) * CH
N16 = N + 16
RP = NT // 16
K1 = int(math.ceil(0.8 * N))
K2 = int(math.ceil(0.8 * K1))
K3 = int(math.ceil(0.8 * K2))
INT_MIN = -2147483648

_mesh = plsc.VectorSubcoreMesh(core_axis_name="c", subcore_axis_name="s")
_sc_params = pltpu.CompilerParams(use_tc_tiling_on_sc=False,
                                  needs_layout_passes=False)


def _make_agg(D, CHD):
    @functools.partial(
        pl.kernel,
        out_type=jax.ShapeDtypeStruct((2, NT, D), jnp.float32),
        mesh=_mesh,
        compiler_params=_sc_params,
        scratch_types=[
            pltpu.VMEM((CHD,), jnp.int32),
            pltpu.VMEM((CHD,), jnp.int32),
            pltpu.VMEM((CHD,), jnp.int32),
            pltpu.VMEM((CHD,), jnp.int32),
            pltpu.VMEM((CHD, D), jnp.float32),
            pltpu.VMEM((CHD, D), jnp.float32),
            pltpu.VMEM((16,), jnp.int32),
            pltpu.VMEM_SHARED((NT, D), jnp.float32),
            pltpu.SemaphoreType.DMA,
            pltpu.SemaphoreType.DMA,
            pltpu.SemaphoreType.DMA,
            pltpu.SemaphoreType.DMA,
        ],
    )
    def agg(table, srcs, dsts, counts, zrows, out, sb0, sb1, db0, db1,
            r0, r1, cbuf, aggs, isem0, isem1, gsem0, gsem1):
        c = lax.axis_index("c")
        s = lax.axis_index("s")
        tile = c * 16 + s
        sb = (sb0, sb1)
        db = (db0, db1)
        rw = (r0, r1)
        isem = (isem0, isem1)
        gsem = (gsem0, gsem1)

        def idx_issue(chv, b):
            pltpu.async_copy(srcs.at[tile, pl.ds(chv * CHD, CHD)], sb[b],
                             isem[b])
            pltpu.async_copy(dsts.at[tile, pl.ds(chv * CHD, CHD)], db[b],
                             isem[b])

        def idx_wait(chv, b):
            pltpu.make_async_copy(srcs.at[tile, pl.ds(chv * CHD, CHD)],
                                  sb[b], isem[b]).wait()
            pltpu.make_async_copy(dsts.at[tile, pl.ds(chv * CHD, CHD)],
                                  db[b], isem[b]).wait()

        pltpu.sync_copy(zrows, r0)
        for i in range(RP // CHD):
            pltpu.sync_copy(r0, aggs.at[pl.ds(s * RP + i * CHD, CHD)])
        plsc.subcore_barrier()
        pltpu.sync_copy(counts.at[tile], cbuf)
        cnt = cbuf[...][0]
        nch = (cnt + (CHD - 1)) // CHD

        @pl.when(nch > 0)
        def _():
            idx_issue(0, 0)
            idx_wait(0, 0)
            pltpu.async_copy(table.at[sb[0]], rw[0], gsem[0])

        @pl.when(nch > 1)
        def _():
            idx_issue(1, 1)

        def body(i2, carry):
            for b in range(2):
                ch = i2 * 2 + b

                @pl.when(ch < nch)
                def _():
                    @pl.when(ch + 1 < nch)
                    def _():
                        idx_wait(ch + 1, 1 - b)
                        pltpu.async_copy(table.at[sb[1 - b]], rw[1 - b],
                                         gsem[1 - b])

                    pltpu.make_async_copy(table.at[sb[b]], rw[b],
                                          gsem[b]).wait()
                    pltpu.sync_copy(rw[b], aggs.at[db[b]], add=True)

                    @pl.when(ch + 2 < nch)
                    def _():
                        idx_issue(ch + 2, b)
            return carry

        lax.fori_loop(0, (nch + 1) // 2, body, jnp.int32(0))
        plsc.subcore_barrier()
        pltpu.sync_copy(aggs.at[pl.ds(s * RP, RP)], out.at[c, pl.ds(s * RP, RP)])

    return agg


@functools.partial(
    pl.kernel,
    out_type=(
        jax.ShapeDtypeStruct((TILES, L), jnp.int32),
        jax.ShapeDtypeStruct((TILES, L), jnp.int32),
        jax.ShapeDtypeStruct((TILES, 16), jnp.int32),
        jax.ShapeDtypeStruct((2, NT, 16), jnp.float32),
    ),
    mesh=_mesh,
    compiler_params=_sc_params,
    scratch_types=[
        pltpu.VMEM((N16,), jnp.int32),
        pltpu.VMEM((L,), jnp.int32),
        pltpu.VMEM((L,), jnp.int32),
        pltpu.VMEM((L,), jnp.int32),
        pltpu.VMEM((L,), jnp.int32),
        pltpu.VMEM((CH,), jnp.int32),
        pltpu.VMEM((CH, 16), jnp.float32),
        pltpu.VMEM((CH, 16), jnp.float32),
        pltpu.VMEM((16,), jnp.int32),
        pltpu.VMEM_SHARED((NT, 16), jnp.float32),
    ],
)
def _filt(srcs, dsts, pids, ones_in, zeros_in, srcf, dstf, counts, degf,
          pidsv, srcv, dstv, sfv, dfv, dstb, onesb, zb, cntb, degs):
    c = lax.axis_index("c")
    s = lax.axis_index("s")
    tile = c * 16 + s
    pltpu.sync_copy(pids, pidsv)
    pltpu.sync_copy(srcs.at[tile], srcv)
    pltpu.sync_copy(dsts.at[tile], dstv)
    pltpu.sync_copy(ones_in, onesb)
    pltpu.sync_copy(zeros_in, zb)
    for i in range(RP // CH):
        pltpu.sync_copy(zb, degs.at[pl.ds(s * RP + i * CH, CH)])
    z16 = jnp.zeros((16,), jnp.int32)
    n16 = jnp.full((16,), N, jnp.int32)

    def pf(i, carry):
        sfv[pl.ds(i * 16, 16)] = z16
        dfv[pl.ds(i * 16, 16)] = n16
        return carry

    lax.fori_loop(0, L // 16, pf, jnp.int32(0))
    plsc.subcore_barrier()

    def cb(i, off):
        s16 = srcv[pl.ds(i * 16, 16)]
        d16 = dstv[pl.ds(i * 16, 16)]
        ps = plsc.load_gather(pidsv, [s16])
        pd = plsc.load_gather(pidsv, [d16])
        m = ps == pd
        plsc.store_compressed(sfv.at[pl.ds(off, 16)], s16, mask=m)
        plsc.store_compressed(dfv.at[pl.ds(off, 16)], d16, mask=m)
        pc = plsc.all_reduce_population_count(m)
        return off + pc[0]

    off = lax.fori_loop(0, L // 16, cb, jnp.int32(0))
    pltpu.sync_copy(sfv, srcf.at[tile])
    pltpu.sync_copy(dfv, dstf.at[tile])
    cntb[...] = jnp.full((16,), 1, jnp.int32) * off
    pltpu.sync_copy(cntb, counts.at[tile])

    def db(i, carry):
        pltpu.sync_copy(dstf.at[tile, pl.ds(i * CH, CH)], dstb)
        pltpu.sync_copy(onesb, degs.at[dstb], add=True)
        return carry

    lax.fori_loop(0, L // CH, db, jnp.int32(0))
    plsc.subcore_barrier()
    pltpu.sync_copy(degs.at[pl.ds(s * RP, RP)], degf.at[c, pl.ds(s * RP, RP)])


_agg32 = _make_agg(32, 128)
_agg128 = _make_agg(128, 128)
_agg144 = _make_agg(144, 64)


def _rowdot(x, w):
    return jnp.sum(x * w[:, 0][None, :], axis=1)


def _gatt_global(x, w, b):
    g = _rowdot(x, w) + b[0]
    gm = jnp.max(g)
    gm = jnp.where(gm > -1e29, gm, 0.0)
    e = jnp.exp(g - gm)
    wts = e / jnp.clip(jnp.sum(e), 1e-16)
    return jnp.sum(wts[:, None] * x, axis=0)[None, :]


def _gatt_p(x, w, b, pids, nmask):
    oh = (pids[:, None] == lax.broadcasted_iota(jnp.int32, (1, P), 1)
          ).astype(jnp.float32)
    g = _rowdot(x, w) + b[0]
    g = jnp.where(nmask > 0, g, -1e30)
    gmax = jnp.max(jnp.where(oh > 0, g[:, None], -jnp.inf), axis=0)
    gmax = jnp.where(gmax > -1e29, gmax, 0.0)
    gpn = jnp.sum(oh * gmax[None, :], axis=1)
    e = jnp.exp(g - gpn) * nmask
    den = jnp.sum(oh * e[:, None], axis=0)
    dpn = jnp.sum(oh * den[None, :], axis=1)
    wts = e / jnp.clip(dpn, 1e-16)
    return lax.dot_general(oh, wts[:, None] * x, (((0,), (0,)), ((), ())),
                           preferred_element_type=jnp.float32)


def _gnorm(x, gnw, gnb, gna):
    mu = jnp.mean(x, axis=0)
    o = x - gna[None, :] * mu[None, :]
    var = jnp.mean(o * o, axis=0)
    return gnw[None, :] * o / jnp.sqrt(var + 1e-5)[None, :] + gnb[None, :]


def _f32key(x):
    i = lax.bitcast_convert_type(x, jnp.int32)
    return jnp.where(i >= 0, i, (~i) ^ jnp.int32(INT_MIN))


def _avg_ceil(lo, hi):
    return (lo >> 1) + (hi >> 1) + ((lo | hi) & 1)


def _topk_mask(sel, k):
    k2d = _f32key(sel).reshape(1, N)
    idx2d = lax.broadcasted_iota(jnp.int32, (1, N), 1)

    def b1(_, carry):
        lo, hi = carry
        mid = _avg_ceil(lo, hi)
        ok = jnp.sum((k2d >= mid).astype(jnp.int32)) >= k
        return jnp.where(ok, mid, lo), jnp.where(ok, hi, mid - 1)

    t, _ = lax.fori_loop(0, 32, b1, (jnp.int32(INT_MIN), jnp.int32(2147483647)))
    need = k - jnp.sum((k2d > t).astype(jnp.int32))

    def b2(_, carry):
        lo, hi = carry
        mid = _avg_ceil(lo, hi)
        ok = jnp.sum(((k2d == t) & (idx2d < mid)).astype(jnp.int32)) >= need
        return jnp.where(ok, lo, mid + 1), jnp.where(ok, mid, hi)

    _, m = lax.fori_loop(0, 15, b2, (jnp.int32(0), jnp.int32(N)))
    sel2d = (k2d > t) | ((k2d == t) & (idx2d < m))
    return sel2d.astype(jnp.float32).reshape(N)


_tc_params = pltpu.CompilerParams(vmem_limit_bytes=100 * 1024 * 1024)


def _tc(fn, out_shape, *args):
    return pl.pallas_call(fn, out_shape=out_shape,
                          compiler_params=_tc_params)(*args)


def _sage_head(agg_self, cnt, x, Wn, bn, Wr):
    mean = agg_self / cnt[:, None]
    return jnp.tanh(
        jnp.dot(mean, Wn, preferred_element_type=jnp.float32)
        + bn[None, :]
        + jnp.dot(x, Wr, preferred_element_type=jnp.float32))


def _tcA_body(agg, x_r, cnt_r, Wn, bn, Wr, gw, gb, gnw, gnb, gna,
              xg_o, xr_o, *, last):
    x = x_r[...]
    a = agg[0, :N, :] + agg[1, :N, :] + x
    xN = _sage_head(a, cnt_r[...], x, Wn[...], bn[...], Wr[...])
    xr_o[...] = _gatt_global(xN, gw[...], gb[...])
    if last:
        xg_o[...] = xN
    else:
        xg_o[...] = _gnorm(xN, gnw[...], gnb[...], gna[...])


def _tc1_body(agg, h_r, Wn, bn, Wr, gw, gb, gnw, gnb, gna, xg_o, xr_o, cnt_o):
    h = h_r[...]
    a = agg[0, :N, :] + agg[1, :N, :]
    cnt = a[:, 16] + 1.0
    aggH = a[:, :IN_DIM] + h
    xA = _sage_head(aggH, cnt, h, Wn[...], bn[...], Wr[...])
    xr_o[...] = _gatt_global(xA, gw[...], gb[...])
    xg_o[...] = _gnorm(xA, gnw[...], gnb[...], gna[...])
    cnt_o[...] = cnt


def _tc4_body(agg, degf, x_r, Wn, bn, Wr, x1B_o):
    x = x_r[...]
    a = agg[0, :N, :] + agg[1, :N, :] + x
    df = degf[0, :N, 0] + degf[1, :N, 0]
    cnt = jnp.clip(df + 1.0, 1.0, None)
    x1B_o[...] = _sage_head(a, cnt, x, Wn[...], bn[...], Wr[...])


def _pool_body(pagg, x_r, mask_prev, wrel, wroot, pb, gw, gb, pids, k,
               t_o, r_o):
    x = x_r[...]
    score = mask_prev * (_rowdot(pagg[0, :N, :], wrel)
                         + _rowdot(pagg[1, :N, :], wrel)
                         + _rowdot(x, wrel)) + _rowdot(x, wroot) + pb[0]
    sel = jnp.where(mask_prev > 0, score, -jnp.inf)
    newmask = _topk_mask(sel, k) * mask_prev
    xP = x * jnp.tanh(score)[:, None] * newmask[:, None]
    r_o[...] = _gatt_p(xP, gw, gb, pids, newmask)
    t_o[...] = jnp.concatenate(
        [xP, jnp.broadcast_to(newmask[:, None], (N, 16))], axis=1)


def _tc5_body(pagg, x_r, wrel, wroot, pb, gw, gb, pids_r, t_o, r_o):
    _pool_body(pagg, x_r, jnp.ones((N,), jnp.float32), wrel[...], wroot[...],
               pb[...], gw[...], gb[...], pids_r[...], K1, t_o, r_o)


def _tc7_body(pagg, x_r, t_r, wrel, wroot, pb, gw, gb, pids_r, t_o, r_o):
    _pool_body(pagg, x_r, t_r[:, H], wrel[...], wroot[...],
               pb[...], gw[...], gb[...], pids_r[...], K2, t_o, r_o)


def _sageB_body(agg, t_r, Wn, bn, Wr, x_o, tp_o):
    xP = t_r[:, :H]
    mprev = t_r[:, H]
    a = agg[0, :N, :] + agg[1, :N, :]
    aggx = mprev[:, None] * a[:, :H] + xP
    cnt = jnp.clip(mprev * a[:, H] + mprev, 1.0, None)
    xN = _sage_head(aggx, cnt, xP, Wn[...], bn[...], Wr[...])
    x_o[...] = xN
    tp_o[...] = xN * mprev[:, None]


def _tc9_body(pagg, x_r, t_r, wrel, wroot, pb, gw, gb, pids_r,
              x1, x2, x3, r1, r2, lin_w, lin_b, m1w, m1b, m2w, m2b, m3w, m3b,
              out_o):
    x = x_r[...]
    mprev = t_r[:, H]
    score = mprev * (_rowdot(pagg[0, :N, :], wrel[...])
                     + _rowdot(pagg[1, :N, :], wrel[...])
                     + _rowdot(x, wrel[...])) + _rowdot(x, wroot[...]) + pb[...][0]
    sel = jnp.where(mprev > 0, score, -jnp.inf)
    m3 = _topk_mask(sel, K3) * mprev
    xP3 = x * jnp.tanh(score)[:, None] * m3[:, None]
    r3 = _gatt_p(xP3, gw[...], gb[...], pids_r[...], m3)
    readout1 = jnp.concatenate([x1[...], x2[...], x3[...]], axis=1)
    readout2 = jnp.concatenate([r1[...], r2[...], r3], axis=1)
    readout = jnp.concatenate([readout1, readout2], axis=0)
    r = jnp.tanh(
        (jnp.dot(readout, lin_w[...], preferred_element_type=jnp.float32)
         + lin_b[...][None, :]).T)
    mm = jnp.tanh(jnp.dot(r, m1w[...], preferred_element_type=jnp.float32)
                  + m1b[...][None, :])
    mm = jnp.tanh(jnp.dot(mm, m2w[...], preferred_element_type=jnp.float32)
                  + m2b[...][None, :])
    mm = jax.nn.sigmoid(jnp.dot(mm, m3w[...], preferred_element_type=jnp.float32)
                        + m3b[...][None, :])
    out_o[...] = jax.nn.sigmoid(mm)


def kernel(h, edge_index, pathway_ids, params):
    f32 = jnp.float32
    pids = pathway_ids.astype(jnp.int32)
    pids_pad = jnp.concatenate([pids, jnp.full((16,), -1, jnp.int32)])
    src = edge_index[0].astype(jnp.int32).reshape(TILES, EPT)
    dst = edge_index[1].astype(jnp.int32).reshape(TILES, EPT)
    src2d = jnp.concatenate(
        [src, jnp.zeros((TILES, L - EPT), jnp.int32)], axis=1)
    dst2d = jnp.concatenate(
        [dst, jnp.full((TILES, L - EPT), N, jnp.int32)], axis=1)
    counts_full = jnp.full((TILES, 16), L, jnp.int32)
    ones16 = jnp.ones((CH, 16), f32)
    z16 = jnp.zeros((CH, 16), f32)
    z32 = jnp.zeros((CH, 32), f32)
    z128 = jnp.zeros((CH, H), f32)
    z144 = jnp.zeros((64, H + 16), f32)
    table_hx = jnp.concatenate([h, jnp.ones((N, 16), f32)], axis=1)

    srcf, dstf, counts, degf = _filt(src2d, dst2d, pids_pad, ones16, z16)
    aggA = _agg32(table_hx, src2d, dst2d, counts_full, z32)

    p = params
    sN = jax.ShapeDtypeStruct((N, H), f32)
    s1 = jax.ShapeDtypeStruct((1, H), f32)
    sP = jax.ShapeDtypeStruct((P, H), f32)
    sT = jax.ShapeDtypeStruct((N, H + 16), f32)

    xg1, x1, cntA = _tc(
        _tc1_body, (sN, s1, jax.ShapeDtypeStruct((N,), f32)),
        aggA, h, p["conva_Wn"], p["conva_bn"], p["conva_Wr"],
        p["gatea_w"], p["gatea_b"], p["gn_w"], p["gn_b"], p["gn_a"])

    aggB = _agg128(xg1, src2d, dst2d, counts_full, z128)
    xg2, x2 = _tc(
        functools.partial(_tcA_body, last=False), (sN, s1),
        aggB, xg1, cntA, p["convb_Wn"], p["convb_bn"], p["convb_Wr"],
        p["gateb_w"], p["gateb_b"], p["gn_w"], p["gn_b"], p["gn_a"])

    aggC = _agg128(xg2, src2d, dst2d, counts_full, z128)
    xC, x3 = _tc(
        functools.partial(_tcA_body, last=True), (sN, s1),
        aggC, xg2, cntA, p["convc_Wn"], p["convc_bn"], p["convc_Wr"],
        p["gatec_w"], p["gatec_b"], p["gn_w"], p["gn_b"], p["gn_a"])

    agg1 = _agg128(xC, srcf, dstf, counts, z128)
    x1B = _tc(_tc4_body, sN, agg1, degf, xC,
              p["conv1_Wn"], p["conv1_bn"], p["conv1_Wr"])

    pagg1 = _agg128(x1B, srcf, dstf, counts, z128)
    t2, r1 = _tc(_tc5_body, (sT, sP), pagg1, x1B,
                 p["pool1_wrel"], p["pool1_wroot"], p["pool1_b"],
                 p["gate1_w"], p["gate1_b"], pids)

    agg2 = _agg144(t2, srcf, dstf, counts, z144)
    x2B, t2p = _tc(_sageB_body, (sN, sN), agg2, t2,
                   p["conv2_Wn"], p["conv2_bn"], p["conv2_Wr"])

    pagg2 = _agg128(t2p, srcf, dstf, counts, z128)
    t3, r2 = _tc(_tc7_body, (sT, sP), pagg2, x2B, t2,
                 p["pool2_wrel"], p["pool2_wroot"], p["pool2_b"],
                 p["gate2_w"], p["gate2_b"], pids)

    agg3 = _agg144(t3, srcf, dstf, counts, z144)
    x3B, t3p = _tc(_sageB_body, (sN, sN), agg3, t3,
                   p["conv3_Wn"], p["conv3_bn"], p["conv3_Wr"])

    pagg3 = _agg128(t3p, srcf, dstf, counts, z128)
    out = _tc(_tc9_body, jax.ShapeDtypeStruct((1, 1), f32),
              pagg3, x3B, t3,
              p["pool3_wrel"], p["pool3_wroot"], p["pool3_b"],
              p["gate3_w"], p["gate3_b"], pids,
              x1, x2, x3, r1, r2,
              p["lin_w"], p["lin_b"], p["mlp1_w"], p["mlp1_b"],
              p["mlp2_w"], p["mlp2_b"], p["mlp3_w"], p["mlp3_b"])
    return out
